# baseline (speedup 1.0000x reference)
import functools

import jax
import jax.numpy as jnp
from jax import lax
from jax.experimental import pallas as pl
from jax.experimental.pallas import tpu as pltpu
from jax.experimental.pallas import tpu_sc as plsc

N_OPS = 100000
EMB_DIM = 16
BATCH = 1024

_NC, _NS = 2, 16
_NW = _NC * _NS
_BPW = BATCH // _NW
_EPW = _BPW * EMB_DIM
_CHUNK = 128


@functools.cache
def _make_sc_gather():
    mesh = plsc.VectorSubcoreMesh(core_axis_name="c", subcore_axis_name="s")

    @functools.partial(
        pl.kernel,
        mesh=mesh,
        out_type=jax.ShapeDtypeStruct((BATCH * EMB_DIM,), jnp.float32),
        scratch_types=[
            pltpu.VMEM((_BPW,), jnp.int32),
            pltpu.VMEM((_EPW,), jnp.int32),
            pltpu.VMEM((_EPW,), jnp.float32),
            pltpu.SemaphoreType.DMA,
        ],
    )
    def _sc_gather(flat_hbm, idx_hbm, out_hbm, idx_v, idxf_v, rows_v, sem):
        wid = lax.axis_index("s") * _NC + lax.axis_index("c")
        base = wid * _BPW
        pltpu.sync_copy(idx_hbm.at[pl.ds(base, _BPW)], idx_v)
        strided = lax.iota(jnp.int32, 16) * N_OPS
        for g in range(_BPW // 16):
            vec = idx_v[pl.ds(g * 16, 16)]
            for k in range(16):
                i = g * 16 + k
                idxf_v[pl.ds(i * EMB_DIM, EMB_DIM)] = vec[k] + strided
        copies = []
        for j in range(_EPW // _CHUNK):
            copies.append(
                pltpu.async_copy(
                    flat_hbm.at[idxf_v.at[pl.ds(j * _CHUNK, _CHUNK)]],
                    rows_v.at[pl.ds(j * _CHUNK, _CHUNK)], sem))
        for c in copies:
            c.wait()
        pltpu.sync_copy(rows_v, out_hbm.at[pl.ds(wid * _EPW, _EPW)])

    return _sc_gather


_TN = 1024


def _mm_body(w_ref, b_ref, h_ref, out_ref):
    waug = jnp.concatenate([w_ref[...], b_ref[...]], axis=0)
    out_ref[...] = lax.dot_general(
        waug.astype(jnp.bfloat16), h_ref[...].astype(jnp.bfloat16),
        (((0,), (1,)), ((), ())),
        preferred_element_type=jnp.float32,
    )


def kernel(x, emb, W, b):
    flat = emb.T.reshape(-1)
    h = _make_sc_gather()(flat, x.astype(jnp.int32)).reshape(BATCH, EMB_DIM)
    haug = jnp.concatenate([h, jnp.ones((BATCH, 1), jnp.float32)], axis=1)
    logits_t = pl.pallas_call(
        _mm_body,
        grid=(pl.cdiv(N_OPS, _TN),),
        in_specs=[
            pl.BlockSpec((EMB_DIM, _TN), lambda j: (0, j)),
            pl.BlockSpec((1, _TN), lambda j: (0, j)),
            pl.BlockSpec((BATCH, EMB_DIM + 1), lambda j: (0, 0)),
        ],
        out_specs=pl.BlockSpec((_TN, BATCH), lambda j: (j, 0)),
        out_shape=jax.ShapeDtypeStruct((N_OPS, BATCH), jnp.float32),
    )(W.T, b[None, :], haug)
    return logits_t.T


# speedup vs baseline: 1.0992x; 1.0992x over previous
import functools

import jax
import jax.numpy as jnp
from jax import lax
from jax.experimental import pallas as pl
from jax.experimental.pallas import tpu as pltpu
from jax.experimental.pallas import tpu_sc as plsc

N_OPS = 100000
EMB_DIM = 16
BATCH = 1024

_NC, _NS = 2, 16
_NW = _NC * _NS
_BPW = BATCH // _NW
_EPW = _BPW * EMB_DIM
_CHUNK = 128


@functools.cache
def _make_sc_gather():
    mesh = plsc.VectorSubcoreMesh(core_axis_name="c", subcore_axis_name="s")

    @functools.partial(
        pl.kernel,
        mesh=mesh,
        out_type=jax.ShapeDtypeStruct((BATCH * EMB_DIM,), jnp.float32),
        scratch_types=[
            pltpu.VMEM((_BPW,), jnp.int32),
            pltpu.VMEM((_EPW,), jnp.int32),
            pltpu.VMEM((_EPW,), jnp.float32),
            pltpu.SemaphoreType.DMA,
        ],
    )
    def _sc_gather(flat_hbm, idx_hbm, out_hbm, idx_v, idxf_v, rows_v, sem):
        wid = lax.axis_index("s") * _NC + lax.axis_index("c")
        base = wid * _BPW
        pltpu.sync_copy(idx_hbm.at[pl.ds(base, _BPW)], idx_v)
        strided = lax.iota(jnp.int32, 16) * N_OPS
        for g in range(_BPW // 16):
            vec = idx_v[pl.ds(g * 16, 16)]
            for k in range(16):
                i = g * 16 + k
                idxf_v[pl.ds(i * EMB_DIM, EMB_DIM)] = vec[k] + strided
        copies = []
        for j in range(_EPW // _CHUNK):
            copies.append(
                pltpu.async_copy(
                    flat_hbm.at[idxf_v.at[pl.ds(j * _CHUNK, _CHUNK)]],
                    rows_v.at[pl.ds(j * _CHUNK, _CHUNK)], sem))
        for c in copies:
            c.wait()
        pltpu.sync_copy(rows_v, out_hbm.at[pl.ds(wid * _EPW, _EPW)])

    return _sc_gather


_TN = 3072


def _mm_body(w_ref, b_ref, h_ref, out_ref):
    waug = jnp.concatenate([w_ref[...], b_ref[...]], axis=0)
    out_ref[...] = lax.dot_general(
        waug.astype(jnp.bfloat16), h_ref[...].astype(jnp.bfloat16),
        (((0,), (1,)), ((), ())),
        preferred_element_type=jnp.float32,
    )


def kernel(x, emb, W, b):
    flat = emb.T.reshape(-1)
    h = _make_sc_gather()(flat, x.astype(jnp.int32)).reshape(BATCH, EMB_DIM)
    haug = jnp.concatenate([h, jnp.ones((BATCH, 1), jnp.float32)], axis=1)
    logits_t = pl.pallas_call(
        _mm_body,
        grid=(pl.cdiv(N_OPS, _TN),),
        in_specs=[
            pl.BlockSpec((EMB_DIM, _TN), lambda j: (0, j)),
            pl.BlockSpec((1, _TN), lambda j: (0, j)),
            pl.BlockSpec((BATCH, EMB_DIM + 1), lambda j: (0, 0)),
        ],
        out_specs=pl.BlockSpec((_TN, BATCH), lambda j: (j, 0)),
        out_shape=jax.ShapeDtypeStruct((N_OPS, BATCH), jnp.float32),
    )(W.T, b[None, :], haug)
    return logits_t.T


# speedup vs baseline: 1.1000x; 1.0008x over previous
import functools

import jax
import jax.numpy as jnp
from jax import lax
from jax.experimental import pallas as pl
from jax.experimental.pallas import tpu as pltpu
from jax.experimental.pallas import tpu_sc as plsc

N_OPS = 100000
EMB_DIM = 16
BATCH = 1024

_NC, _NS = 2, 16
_NW = _NC * _NS
_BPW = BATCH // _NW
_EPW = _BPW * EMB_DIM
_CHUNK = 128


@functools.cache
def _make_sc_gather():
    mesh = plsc.VectorSubcoreMesh(core_axis_name="c", subcore_axis_name="s")

    @functools.partial(
        pl.kernel,
        mesh=mesh,
        out_type=jax.ShapeDtypeStruct((BATCH * EMB_DIM,), jnp.float32),
        scratch_types=[
            pltpu.VMEM((_BPW,), jnp.int32),
            pltpu.VMEM((_EPW,), jnp.int32),
            pltpu.VMEM((_EPW,), jnp.float32),
            pltpu.SemaphoreType.DMA,
        ],
    )
    def _sc_gather(flat_hbm, idx_hbm, out_hbm, idx_v, idxf_v, rows_v, sem):
        wid = lax.axis_index("s") * _NC + lax.axis_index("c")
        base = wid * _BPW
        pltpu.sync_copy(idx_hbm.at[pl.ds(base, _BPW)], idx_v)
        strided = lax.iota(jnp.int32, 16) * N_OPS
        for g in range(_BPW // 16):
            vec = idx_v[pl.ds(g * 16, 16)]
            for k in range(16):
                i = g * 16 + k
                idxf_v[pl.ds(i * EMB_DIM, EMB_DIM)] = vec[k] + strided
        copies = []
        for j in range(_EPW // _CHUNK):
            copies.append(
                pltpu.async_copy(
                    flat_hbm.at[idxf_v.at[pl.ds(j * _CHUNK, _CHUNK)]],
                    rows_v.at[pl.ds(j * _CHUNK, _CHUNK)], sem))
        for c in copies:
            c.wait()
        pltpu.sync_copy(rows_v, out_hbm.at[pl.ds(wid * _EPW, _EPW)])

    return _sc_gather


_TN = 2048


def _mm_body(w_ref, b_ref, h_ref, out_ref):
    waug = jnp.concatenate([w_ref[...], b_ref[...]], axis=0)
    out_ref[...] = lax.dot_general(
        waug.astype(jnp.bfloat16), h_ref[...].astype(jnp.bfloat16),
        (((0,), (1,)), ((), ())),
        preferred_element_type=jnp.float32,
    )


def kernel(x, emb, W, b):
    flat = emb.T.reshape(-1)
    h = _make_sc_gather()(flat, x.astype(jnp.int32)).reshape(BATCH, EMB_DIM)
    haug = jnp.concatenate([h, jnp.ones((BATCH, 1), jnp.float32)], axis=1)
    logits_t = pl.pallas_call(
        _mm_body,
        grid=(pl.cdiv(N_OPS, _TN),),
        in_specs=[
            pl.BlockSpec((EMB_DIM, _TN), lambda j: (0, j)),
            pl.BlockSpec((1, _TN), lambda j: (0, j)),
            pl.BlockSpec((BATCH, EMB_DIM + 1), lambda j: (0, 0)),
        ],
        out_specs=pl.BlockSpec((_TN, BATCH), lambda j: (j, 0)),
        out_shape=jax.ShapeDtypeStruct((N_OPS, BATCH), jnp.float32),
    )(W.T, b[None, :], haug)
    return logits_t.T


# speedup vs baseline: 1.1081x; 1.0073x over previous
import functools

import jax
import jax.numpy as jnp
from jax import lax
from jax.experimental import pallas as pl
from jax.experimental.pallas import tpu as pltpu
from jax.experimental.pallas import tpu_sc as plsc

N_OPS = 100000
EMB_DIM = 16
BATCH = 1024

_STRIDE = N_OPS
_NC, _NS = 2, 16
_NW = _NC * _NS
_BPW = BATCH // _NW
_EPW = _BPW * EMB_DIM
_CHUNK = 128


@functools.cache
def _make_sc_gather():
    mesh = plsc.VectorSubcoreMesh(core_axis_name="c", subcore_axis_name="s")

    @functools.partial(
        pl.kernel,
        mesh=mesh,
        out_type=jax.ShapeDtypeStruct((BATCH, EMB_DIM + 1), jnp.float32),
        scratch_types=[
            pltpu.VMEM((_BPW,), jnp.int32),
            pltpu.VMEM((_EPW,), jnp.int32),
            pltpu.VMEM((_EPW,), jnp.float32),
            pltpu.VMEM((_BPW, EMB_DIM + 1), jnp.float32),
            pltpu.SemaphoreType.DMA,
        ],
    )
    def _sc_gather(flat_hbm, idx_hbm, out_hbm, idx_v, idxf_v, rows_v,
                   haug_v, sem):
        wid = lax.axis_index("s") * _NC + lax.axis_index("c")
        base = wid * _BPW
        pltpu.sync_copy(idx_hbm.at[pl.ds(base, _BPW)], idx_v)
        strided = lax.iota(jnp.int32, 16) * _STRIDE
        for g in range(_BPW // 16):
            vec = idx_v[pl.ds(g * 16, 16)]
            for k in range(16):
                i = g * 16 + k
                idxf_v[pl.ds(i * EMB_DIM, EMB_DIM)] = vec[k] + strided
        copies = []
        for j in range(_EPW // _CHUNK):
            copies.append(
                pltpu.async_copy(
                    flat_hbm.at[idxf_v.at[pl.ds(j * _CHUNK, _CHUNK)]],
                    rows_v.at[pl.ds(j * _CHUNK, _CHUNK)], sem))
        ones = jnp.ones((16,), jnp.float32)
        for i in range(_BPW):
            haug_v[i, pl.ds(0, 16)] = ones
        for c in copies:
            c.wait()
        for i in range(_BPW):
            haug_v[i, pl.ds(1, EMB_DIM)] = rows_v[pl.ds(i * EMB_DIM, EMB_DIM)]
        pltpu.sync_copy(haug_v, out_hbm.at[pl.ds(base, _BPW)])

    return _sc_gather


_TN = 2048


def _mm_body(b_ref, w_ref, h_ref, out_ref):
    waug = jnp.concatenate([b_ref[...], w_ref[...]], axis=0)
    out_ref[...] = lax.dot_general(
        waug.astype(jnp.bfloat16), h_ref[...].astype(jnp.bfloat16),
        (((0,), (1,)), ((), ())),
        preferred_element_type=jnp.float32,
    )


def kernel(x, emb, W, b):
    flat = emb.T.reshape(-1)
    haug = _make_sc_gather()(flat, x.astype(jnp.int32))
    logits_t = pl.pallas_call(
        _mm_body,
        grid=(pl.cdiv(N_OPS, _TN),),
        in_specs=[
            pl.BlockSpec((1, _TN), lambda j: (0, j)),
            pl.BlockSpec((EMB_DIM, _TN), lambda j: (0, j)),
            pl.BlockSpec((BATCH, EMB_DIM + 1), lambda j: (0, 0)),
        ],
        out_specs=pl.BlockSpec((_TN, BATCH), lambda j: (j, 0)),
        out_shape=jax.ShapeDtypeStruct((N_OPS, BATCH), jnp.float32),
    )(b[None, :], W.T, haug)
    return logits_t.T


# speedup vs baseline: 1.1101x; 1.0019x over previous
import functools

import jax
import jax.numpy as jnp
from jax import lax
from jax.experimental import pallas as pl
from jax.experimental.pallas import tpu as pltpu
from jax.experimental.pallas import tpu_sc as plsc

N_OPS = 100000
EMB_DIM = 16
BATCH = 1024

_STRIDE = N_OPS
_NC, _NS = 2, 16
_NW = _NC * _NS
_BPW = BATCH // _NW
_EPW = _BPW * EMB_DIM
_CHUNK = 128


@functools.cache
def _make_sc_gather():
    mesh = plsc.VectorSubcoreMesh(core_axis_name="c", subcore_axis_name="s")

    @functools.partial(
        pl.kernel,
        mesh=mesh,
        out_type=jax.ShapeDtypeStruct((BATCH, EMB_DIM + 1), jnp.float32),
        scratch_types=[
            pltpu.VMEM((_BPW,), jnp.int32),
            pltpu.VMEM((_EPW,), jnp.int32),
            pltpu.VMEM((_EPW,), jnp.float32),
            pltpu.VMEM((_BPW, EMB_DIM + 1), jnp.float32),
            pltpu.SemaphoreType.DMA,
        ],
    )
    def _sc_gather(flat_hbm, idx_hbm, out_hbm, idx_v, idxf_v, rows_v,
                   haug_v, sem):
        wid = lax.axis_index("s") * _NC + lax.axis_index("c")
        base = wid * _BPW
        pltpu.sync_copy(idx_hbm.at[pl.ds(base, _BPW)], idx_v)
        strided = lax.iota(jnp.int32, 16) * _STRIDE
        for g in range(_BPW // 16):
            vec = idx_v[pl.ds(g * 16, 16)]
            for k in range(16):
                i = g * 16 + k
                idxf_v[pl.ds(i * EMB_DIM, EMB_DIM)] = vec[k] + strided
        copies = []
        for j in range(_EPW // _CHUNK):
            copies.append(
                pltpu.async_copy(
                    flat_hbm.at[idxf_v.at[pl.ds(j * _CHUNK, _CHUNK)]],
                    rows_v.at[pl.ds(j * _CHUNK, _CHUNK)], sem))
        ones = jnp.ones((16,), jnp.float32)
        for i in range(_BPW):
            haug_v[i, pl.ds(0, 16)] = ones
        for c in copies:
            c.wait()
        for i in range(_BPW):
            haug_v[i, pl.ds(1, EMB_DIM)] = rows_v[pl.ds(i * EMB_DIM, EMB_DIM)]
        pltpu.sync_copy(haug_v, out_hbm.at[pl.ds(base, _BPW)])

    return _sc_gather


_TN = 2048


def _mm_body(b_ref, w_ref, h_ref, out_ref):
    waug = jnp.concatenate([b_ref[...], w_ref[...]], axis=0)
    out_ref[...] = lax.dot_general(
        waug.astype(jnp.bfloat16), h_ref[...].astype(jnp.bfloat16),
        (((0,), (1,)), ((), ())),
        preferred_element_type=jnp.float32,
    )


def kernel(x, emb, W, b):
    flat = emb.T.reshape(-1)
    haug = _make_sc_gather()(flat, x.astype(jnp.int32))
    logits_t = pl.pallas_call(
        _mm_body,
        grid=(pl.cdiv(N_OPS, _TN),),
        in_specs=[
            pl.BlockSpec((1, _TN), lambda j: (0, j)),
            pl.BlockSpec((EMB_DIM, _TN), lambda j: (0, j)),
            pl.BlockSpec((BATCH, EMB_DIM + 1), lambda j: (0, 0)),
        ],
        out_specs=pl.BlockSpec((_TN, BATCH), lambda j: (j, 0)),
        out_shape=jax.ShapeDtypeStruct((N_OPS, BATCH), jnp.float32),
        compiler_params=pltpu.CompilerParams(
            dimension_semantics=("parallel",)),
    )(b[None, :], W.T, haug)
    return logits_t.T


# speedup vs baseline: 1.1107x; 1.0005x over previous
import functools

import jax
import jax.numpy as jnp
from jax import lax
from jax.experimental import pallas as pl
from jax.experimental.pallas import tpu as pltpu
from jax.experimental.pallas import tpu_sc as plsc

N_OPS = 100000
EMB_DIM = 16
BATCH = 1024

_STRIDE = N_OPS
_NC, _NS = 2, 16
_NW = _NC * _NS
_BPW = BATCH // _NW
_EPW = _BPW * EMB_DIM
_CHUNK = 128


@functools.cache
def _make_sc_gather():
    mesh = plsc.VectorSubcoreMesh(core_axis_name="c", subcore_axis_name="s")

    @functools.partial(
        pl.kernel,
        mesh=mesh,
        out_type=jax.ShapeDtypeStruct((BATCH, EMB_DIM + 1), jnp.float32),
        scratch_types=[
            pltpu.VMEM((_BPW,), jnp.int32),
            pltpu.VMEM((_EPW,), jnp.int32),
            pltpu.VMEM((_EPW,), jnp.float32),
            pltpu.VMEM((_BPW, EMB_DIM + 1), jnp.float32),
            pltpu.SemaphoreType.DMA,
        ],
    )
    def _sc_gather(flat_hbm, idx_hbm, out_hbm, idx_v, idxf_v, rows_v,
                   haug_v, sem):
        wid = lax.axis_index("s") * _NC + lax.axis_index("c")
        base = wid * _BPW
        pltpu.sync_copy(idx_hbm.at[pl.ds(base, _BPW)], idx_v)
        strided = lax.iota(jnp.int32, 16) * _STRIDE
        for g in range(_BPW // 16):
            vec = idx_v[pl.ds(g * 16, 16)]
            for k in range(16):
                i = g * 16 + k
                idxf_v[pl.ds(i * EMB_DIM, EMB_DIM)] = vec[k] + strided
        copies = []
        for j in range(_EPW // _CHUNK):
            copies.append(
                pltpu.async_copy(
                    flat_hbm.at[idxf_v.at[pl.ds(j * _CHUNK, _CHUNK)]],
                    rows_v.at[pl.ds(j * _CHUNK, _CHUNK)], sem))
        ones = jnp.ones((16,), jnp.float32)
        for i in range(_BPW):
            haug_v[i, pl.ds(0, 16)] = ones
        for c in copies:
            c.wait()
        for i in range(_BPW):
            haug_v[i, pl.ds(1, EMB_DIM)] = rows_v[pl.ds(i * EMB_DIM, EMB_DIM)]
        pltpu.sync_copy(haug_v, out_hbm.at[pl.ds(base, _BPW)])

    return _sc_gather


_TN = 2048


def _mm_body(b_ref, w_ref, h_ref, out_ref):
    waug = jnp.concatenate([b_ref[...], w_ref[...]], axis=0)
    out_ref[...] = lax.dot_general(
        waug.astype(jnp.bfloat16), h_ref[...].astype(jnp.bfloat16),
        (((0,), (1,)), ((), ())),
        preferred_element_type=jnp.float32,
    )


def kernel(x, emb, W, b):
    flat = emb.T.reshape(-1)
    haug = _make_sc_gather()(flat, x.astype(jnp.int32))
    logits_t = pl.pallas_call(
        _mm_body,
        grid=(pl.cdiv(N_OPS, _TN),),
        in_specs=[
            pl.BlockSpec((1, _TN), lambda j: (0, j)),
            pl.BlockSpec((EMB_DIM, _TN), lambda j: (0, j)),
            pl.BlockSpec((BATCH, EMB_DIM + 1), lambda j: (0, 0)),
        ],
        out_specs=pl.BlockSpec((_TN, BATCH), lambda j: (j, 0)),
        out_shape=jax.ShapeDtypeStruct((N_OPS, BATCH), jnp.float32),
    )(b[None, :], W.T, haug)
    return logits_t.T
